# baseline (speedup 1.0000x reference)
import functools

import jax
import jax.numpy as jnp
from jax import lax
from jax.experimental import pallas as pl
from jax.experimental.pallas import tpu as pltpu
from jax.experimental.pallas import tpu_sc as plsc

N = 10000
D = 128
E = 320000
NC = 2
NS = 16
NW = NC * NS
EPW = E // NW
CH = 125
NCHUNK = EPW // CH
SRPT = 624
TAILO = NS * SRPT
TAIL = N - TAILO
BN = 1000

_MESH = plsc.VectorSubcoreMesh(core_axis_name="c", subcore_axis_name="s")


DW = 16


HC = NCHUNK // 2


def _make_agg(W):
    @functools.partial(
        pl.kernel,
        mesh=_MESH,
        out_type=jax.ShapeDtypeStruct((NC * N, W), jnp.float32),
        scratch_types=[
            pltpu.VMEM((HC, CH), jnp.int32),
            pltpu.VMEM((HC, CH), jnp.int32),
            pltpu.VMEM((CH, W), jnp.float32),
            pltpu.VMEM((CH, W), jnp.float32),
            pltpu.VMEM_SHARED((N, W), jnp.float32),
            pltpu.SemaphoreType.DMA,
            pltpu.SemaphoreType.DMA,
        ],
    )
    def agg(y_hbm, srcs_hbm, dsts_hbm, zero_hbm, out_hbm,
            src_v, dst_v, rows_a, rows_b, acc_sh, sem_a, sem_b):
        cid = lax.axis_index("c")
        sid = lax.axis_index("s")
        wid = sid * NC + cid
        pltpu.sync_copy(zero_hbm.at[pl.ds(sid * SRPT, SRPT)],
                        acc_sh.at[pl.ds(sid * SRPT, SRPT)])

        @pl.when(sid == 0)
        def _():
            pltpu.sync_copy(zero_hbm.at[pl.ds(TAILO, TAIL)],
                            acc_sh.at[pl.ds(TAILO, TAIL)])

        plsc.subcore_barrier()

        for half in range(2):
            pltpu.sync_copy(srcs_hbm.at[wid, pl.ds(half * HC, HC)], src_v)
            pltpu.sync_copy(dsts_hbm.at[wid, pl.ds(half * HC, HC)], dst_v)
            pltpu.async_copy(y_hbm.at[src_v.at[0]], rows_a, sem_a)

            def body(jj, carry):
                j = jj * 2
                pltpu.async_copy(y_hbm.at[src_v.at[j + 1]], rows_b, sem_b)
                pltpu.make_async_copy(y_hbm.at[src_v.at[j]], rows_a,
                                      sem_a).wait()
                pltpu.sync_copy(rows_a, acc_sh.at[dst_v.at[j]], add=True)

                @pl.when(j + 2 < HC)
                def _():
                    pltpu.async_copy(y_hbm.at[src_v.at[j + 2]], rows_a, sem_a)

                pltpu.make_async_copy(y_hbm.at[src_v.at[j + 1]], rows_b,
                                      sem_b).wait()
                pltpu.sync_copy(rows_b, acc_sh.at[dst_v.at[j + 1]], add=True)
                return carry

            lax.fori_loop(0, HC // 2, body, 0)

        plsc.subcore_barrier()
        pltpu.sync_copy(acc_sh.at[pl.ds(sid * SRPT, SRPT)],
                        out_hbm.at[pl.ds(cid * N + sid * SRPT, SRPT)])

        @pl.when(sid == 0)
        def _():
            pltpu.sync_copy(acc_sh.at[pl.ds(TAILO, TAIL)],
                            out_hbm.at[pl.ds(cid * N + TAILO, TAIL)])

    return agg


_sc_agg = _make_agg(D)


@functools.partial(
    pl.kernel,
    mesh=_MESH,
    out_type=jax.ShapeDtypeStruct((NC * N, D), jnp.float32),
    scratch_types=[
        pltpu.VMEM((NCHUNK, CH), jnp.int32),
        pltpu.VMEM((CH, D), jnp.float32),
        pltpu.VMEM_SHARED((N, D), jnp.float32),
    ],
)
def _sc_deg(dsts_hbm, ones_hbm, zero_hbm, out_hbm, dst_v, ones_v, acc_sh):
    cid = lax.axis_index("c")
    sid = lax.axis_index("s")
    wid = sid * NC + cid
    pltpu.sync_copy(zero_hbm.at[pl.ds(sid * SRPT, SRPT)],
                    acc_sh.at[pl.ds(sid * SRPT, SRPT)])

    @pl.when(sid == 0)
    def _():
        pltpu.sync_copy(zero_hbm.at[pl.ds(TAILO, TAIL)],
                        acc_sh.at[pl.ds(TAILO, TAIL)])

    pltpu.sync_copy(dsts_hbm.at[wid], dst_v)
    pltpu.sync_copy(ones_hbm, ones_v)
    plsc.subcore_barrier()

    def body(j, carry):
        pltpu.sync_copy(ones_v, acc_sh.at[dst_v.at[j]], add=True)
        return carry

    lax.fori_loop(0, NCHUNK, body, 0)
    plsc.subcore_barrier()
    pltpu.sync_copy(acc_sh.at[pl.ds(sid * SRPT, SRPT)],
                    out_hbm.at[pl.ds(cid * N + sid * SRPT, SRPT)])

    @pl.when(sid == 0)
    def _():
        pltpu.sync_copy(acc_sh.at[pl.ds(TAILO, TAIL)],
                        out_hbm.at[pl.ds(cid * N + TAILO, TAIL)])


def _mean_block(p_ref, deg_ref):
    agg = p_ref[0] + p_ref[1]
    d = deg_ref[0, :, 0:1] + deg_ref[1, :, 0:1]
    return agg * (1.0 / jnp.maximum(d, 1.0))


def _layer0_body(p_ref, deg_ref, h_ref,
                 wl1_ref, wr1_ref, b1_ref, wl2_ref, wr2_ref, b2_ref,
                 o1_ref, o2_ref):
    mean = _mean_block(p_ref, deg_ref)
    h = h_ref[...]
    for wl, wr, b, o in ((wl1_ref, wr1_ref, b1_ref, o1_ref),
                         (wl2_ref, wr2_ref, b2_ref, o2_ref)):
        v = (jnp.dot(mean, wl[...], preferred_element_type=jnp.float32)
             + jnp.dot(h, wr[...], preferred_element_type=jnp.float32)
             + b[...])
        o[...] = jnp.maximum(v, 0.0)


def _layer_res_body(p1_ref, p2_ref, deg_ref, h1_ref, h2_ref,
                    wl1_ref, wr1_ref, b1_ref, rw1_ref, rb1_ref,
                    wl2_ref, wr2_ref, b2_ref, rw2_ref, rb2_ref,
                    o1_ref, o2_ref, *, relu):
    for p, h_ref, wl, wr, b, rw, rb, o in (
            (p1_ref, h1_ref, wl1_ref, wr1_ref, b1_ref, rw1_ref, rb1_ref,
             o1_ref),
            (p2_ref, h2_ref, wl2_ref, wr2_ref, b2_ref, rw2_ref, rb2_ref,
             o2_ref)):
        mean = _mean_block(p, deg_ref)
        h = h_ref[...]
        v = (jnp.dot(mean, wl[...], preferred_element_type=jnp.float32)
             + jnp.dot(h, wr[...], preferred_element_type=jnp.float32)
             + b[...]
             + jnp.dot(h, rw[...], preferred_element_type=jnp.float32)
             + rb[...])
        o[...] = jnp.maximum(v, 0.0) if relu else v


def _mm2_body(h1_ref, h2_ref, w1_ref, w2_ref, o_ref):
    o_ref[...] = (
        jnp.dot(h1_ref[...], w1_ref[...], preferred_element_type=jnp.float32)
        + jnp.dot(h2_ref[...], w2_ref[...], preferred_element_type=jnp.float32))


def _final_body(p_ref, deg_ref, h1_ref, h2_ref, wr1_ref, wr2_ref, b_ref, o_ref):
    o = (_mean_block(p_ref, deg_ref)
         + jnp.dot(h1_ref[...], wr1_ref[...], preferred_element_type=jnp.float32)
         + jnp.dot(h2_ref[...], wr2_ref[...], preferred_element_type=jnp.float32)
         + b_ref[...])
    m = jnp.max(o, axis=1, keepdims=True)
    e = o - m
    lse = jnp.log(jnp.sum(jnp.exp(e), axis=1, keepdims=True))
    o_ref[...] = e - lse


_P_SPEC = pl.BlockSpec((2, BN, D), lambda i: (0, i, 0))
_DEG_SPEC = pl.BlockSpec((2, BN, DW), lambda i: (0, i, 0))
_H_SPEC = pl.BlockSpec((BN, D), lambda i: (i, 0))
_W_SPEC = pl.BlockSpec((D, D), lambda i: (0, 0))
_B_SPEC = pl.BlockSpec((1, D), lambda i: (0, 0))
_OUT_SHAPE = jax.ShapeDtypeStruct((N, D), jnp.float32)


_layer0 = pl.pallas_call(
    _layer0_body, grid=(N // BN,),
    in_specs=[_P_SPEC, _DEG_SPEC, _H_SPEC,
              _W_SPEC, _W_SPEC, _B_SPEC, _W_SPEC, _W_SPEC, _B_SPEC],
    out_specs=[_H_SPEC, _H_SPEC], out_shape=[_OUT_SHAPE, _OUT_SHAPE])

_RES_SPECS = [_P_SPEC, _P_SPEC, _DEG_SPEC, _H_SPEC, _H_SPEC,
              _W_SPEC, _W_SPEC, _B_SPEC, _W_SPEC, _B_SPEC,
              _W_SPEC, _W_SPEC, _B_SPEC, _W_SPEC, _B_SPEC]

_layer_res_relu = pl.pallas_call(
    functools.partial(_layer_res_body, relu=True), grid=(N // BN,),
    in_specs=_RES_SPECS,
    out_specs=[_H_SPEC, _H_SPEC], out_shape=[_OUT_SHAPE, _OUT_SHAPE])

_layer_res = pl.pallas_call(
    functools.partial(_layer_res_body, relu=False), grid=(N // BN,),
    in_specs=_RES_SPECS,
    out_specs=[_H_SPEC, _H_SPEC], out_shape=[_OUT_SHAPE, _OUT_SHAPE])

_mm2 = pl.pallas_call(
    _mm2_body, grid=(N // BN,),
    in_specs=[_H_SPEC, _H_SPEC, _W_SPEC, _W_SPEC],
    out_specs=_H_SPEC, out_shape=_OUT_SHAPE)

_final = pl.pallas_call(
    _final_body, grid=(N // BN,),
    in_specs=[_P_SPEC, _DEG_SPEC, _H_SPEC, _H_SPEC, _W_SPEC, _W_SPEC, _B_SPEC],
    out_specs=_H_SPEC, out_shape=_OUT_SHAPE)


def kernel(x, edge_index, params):
    src = edge_index[0]
    dst = edge_index[1]
    srcs = src.reshape(NW, NCHUNK, CH)
    dsts = dst.reshape(NW, NCHUNK, CH)
    zero = jnp.zeros((N, D), jnp.float32)
    ones_ch = jnp.ones((CH, D), jnp.float32)

    def agg(y):
        return _sc_agg(y, srcs, dsts, zero).reshape(2, N, D)

    degp = _sc_deg(dsts, ones_ch, zero).reshape(2, N, D)[:, :, :DW]
    p0 = agg(x)
    h1, h2 = _layer0(p0, degp, x,
                     params["c1_Wl0"], params["c1_Wr0"],
                     params["c1_bl0"].reshape(1, D),
                     params["c2_Wl0"], params["c2_Wr0"],
                     params["c2_bl0"].reshape(1, D))
    for i in range(1, 4):
        layer = _layer_res_relu if i < 3 else _layer_res
        p1 = agg(h1)
        p2 = agg(h2)
        h1, h2 = layer(p1, p2, degp, h1, h2,
                       params[f"c1_Wl{i}"], params[f"c1_Wr{i}"],
                       params[f"c1_bl{i}"].reshape(1, D),
                       params[f"c1_Rw{i-1}"],
                       params[f"c1_Rb{i-1}"].reshape(1, D),
                       params[f"c2_Wl{i}"], params[f"c2_Wr{i}"],
                       params[f"c2_bl{i}"].reshape(1, D),
                       params[f"c2_Rw{i-1}"],
                       params[f"c2_Rb{i-1}"].reshape(1, D))

    f_Wl = params["f_Wl"]
    f_Wr = params["f_Wr"]
    y = _mm2(h1, h2, f_Wl[:D], f_Wl[D:])
    pf = agg(y)
    return _final(pf, degp, h1, h2, f_Wr[:D], f_Wr[D:],
                  params["f_bl"].reshape(1, D))


# speedup vs baseline: 1.0281x; 1.0281x over previous
import functools

import jax
import jax.numpy as jnp
from jax import lax
from jax.experimental import pallas as pl
from jax.experimental.pallas import tpu as pltpu
from jax.experimental.pallas import tpu_sc as plsc

N = 10000
D = 128
E = 320000
NC = 2
NS = 16
NW = NC * NS
EPW = E // NW
CH = 125
NCHUNK = EPW // CH
SRPT = 624
TAILO = NS * SRPT
TAIL = N - TAILO
BN = 1000

_MESH = plsc.VectorSubcoreMesh(core_axis_name="c", subcore_axis_name="s")


DW = 16


HC = NCHUNK // 2


def _make_agg(W):
    @functools.partial(
        pl.kernel,
        mesh=_MESH,
        out_type=jax.ShapeDtypeStruct((NC * N, W), jnp.float32),
        scratch_types=[
            pltpu.VMEM((HC, CH), jnp.int32),
            pltpu.VMEM((HC, CH), jnp.int32),
            pltpu.VMEM((CH, W), jnp.float32),
            pltpu.VMEM((CH, W), jnp.float32),
            pltpu.VMEM_SHARED((N, W), jnp.float32),
            pltpu.SemaphoreType.DMA,
            pltpu.SemaphoreType.DMA,
        ],
    )
    def agg(y_hbm, srcs_hbm, dsts_hbm, zero_hbm, out_hbm,
            src_v, dst_v, rows_a, rows_b, acc_sh, sem_a, sem_b):
        cid = lax.axis_index("c")
        sid = lax.axis_index("s")
        wid = sid * NC + cid
        pltpu.sync_copy(zero_hbm.at[pl.ds(sid * SRPT, SRPT)],
                        acc_sh.at[pl.ds(sid * SRPT, SRPT)])

        @pl.when(sid == 0)
        def _():
            pltpu.sync_copy(zero_hbm.at[pl.ds(TAILO, TAIL)],
                            acc_sh.at[pl.ds(TAILO, TAIL)])

        plsc.subcore_barrier()

        for half in range(2):
            pltpu.sync_copy(srcs_hbm.at[wid, pl.ds(half * HC, HC)], src_v)
            pltpu.sync_copy(dsts_hbm.at[wid, pl.ds(half * HC, HC)], dst_v)
            pltpu.async_copy(y_hbm.at[src_v.at[0]], rows_a, sem_a)

            def body(jj, carry):
                j = jj * 2
                pltpu.async_copy(y_hbm.at[src_v.at[j + 1]], rows_b, sem_b)
                pltpu.make_async_copy(y_hbm.at[src_v.at[j]], rows_a,
                                      sem_a).wait()
                pltpu.sync_copy(rows_a, acc_sh.at[dst_v.at[j]], add=True)

                @pl.when(j + 2 < HC)
                def _():
                    pltpu.async_copy(y_hbm.at[src_v.at[j + 2]], rows_a, sem_a)

                pltpu.make_async_copy(y_hbm.at[src_v.at[j + 1]], rows_b,
                                      sem_b).wait()
                pltpu.sync_copy(rows_b, acc_sh.at[dst_v.at[j + 1]], add=True)
                return carry

            lax.fori_loop(0, HC // 2, body, 0)

        plsc.subcore_barrier()
        pltpu.sync_copy(acc_sh.at[pl.ds(sid * SRPT, SRPT)],
                        out_hbm.at[pl.ds(cid * N + sid * SRPT, SRPT)])

        @pl.when(sid == 0)
        def _():
            pltpu.sync_copy(acc_sh.at[pl.ds(TAILO, TAIL)],
                            out_hbm.at[pl.ds(cid * N + TAILO, TAIL)])

    return agg


_sc_agg = _make_agg(D)


@functools.partial(
    pl.kernel,
    mesh=_MESH,
    out_type=jax.ShapeDtypeStruct((NC * N, D), jnp.float32),
    scratch_types=[
        pltpu.VMEM((NCHUNK, CH), jnp.int32),
        pltpu.VMEM((CH, D), jnp.float32),
        pltpu.VMEM_SHARED((N, D), jnp.float32),
    ],
)
def _sc_deg(dsts_hbm, ones_hbm, zero_hbm, out_hbm, dst_v, ones_v, acc_sh):
    cid = lax.axis_index("c")
    sid = lax.axis_index("s")
    wid = sid * NC + cid
    pltpu.sync_copy(zero_hbm.at[pl.ds(sid * SRPT, SRPT)],
                    acc_sh.at[pl.ds(sid * SRPT, SRPT)])

    @pl.when(sid == 0)
    def _():
        pltpu.sync_copy(zero_hbm.at[pl.ds(TAILO, TAIL)],
                        acc_sh.at[pl.ds(TAILO, TAIL)])

    pltpu.sync_copy(dsts_hbm.at[wid], dst_v)
    pltpu.sync_copy(ones_hbm, ones_v)
    plsc.subcore_barrier()

    def body(j, carry):
        pltpu.sync_copy(ones_v, acc_sh.at[dst_v.at[j]], add=True)
        return carry

    lax.fori_loop(0, NCHUNK, body, 0)
    plsc.subcore_barrier()
    pltpu.sync_copy(acc_sh.at[pl.ds(sid * SRPT, SRPT)],
                    out_hbm.at[pl.ds(cid * N + sid * SRPT, SRPT)])

    @pl.when(sid == 0)
    def _():
        pltpu.sync_copy(acc_sh.at[pl.ds(TAILO, TAIL)],
                        out_hbm.at[pl.ds(cid * N + TAILO, TAIL)])


def _mean_block(p_ref, deg_ref):
    agg = p_ref[0] + p_ref[1]
    d = deg_ref[0, :, 0:1] + deg_ref[1, :, 0:1]
    return agg * (1.0 / jnp.maximum(d, 1.0))


def _layer0_body(p_ref, deg_ref, h_ref, wl_ref, wr_ref, b_ref, o_ref):
    mean = _mean_block(p_ref, deg_ref)
    o = (jnp.dot(mean, wl_ref[...], preferred_element_type=jnp.float32)
         + jnp.dot(h_ref[...], wr_ref[...], preferred_element_type=jnp.float32)
         + b_ref[...])
    o_ref[...] = jnp.maximum(o, 0.0)


def _layer_res_body(p_ref, deg_ref, h_ref, wl_ref, wr_ref, b_ref,
                    rw_ref, rb_ref, o_ref, *, relu):
    mean = _mean_block(p_ref, deg_ref)
    h = h_ref[...]
    o = (jnp.dot(mean, wl_ref[...], preferred_element_type=jnp.float32)
         + jnp.dot(h, wr_ref[...], preferred_element_type=jnp.float32)
         + b_ref[...]
         + jnp.dot(h, rw_ref[...], preferred_element_type=jnp.float32)
         + rb_ref[...])
    o_ref[...] = jnp.maximum(o, 0.0) if relu else o


def _mm2_body(h1_ref, h2_ref, w1_ref, w2_ref, o_ref):
    o_ref[...] = (
        jnp.dot(h1_ref[...], w1_ref[...], preferred_element_type=jnp.float32)
        + jnp.dot(h2_ref[...], w2_ref[...], preferred_element_type=jnp.float32))


def _final_body(p_ref, deg_ref, h1_ref, h2_ref, wr1_ref, wr2_ref, b_ref, o_ref):
    o = (_mean_block(p_ref, deg_ref)
         + jnp.dot(h1_ref[...], wr1_ref[...], preferred_element_type=jnp.float32)
         + jnp.dot(h2_ref[...], wr2_ref[...], preferred_element_type=jnp.float32)
         + b_ref[...])
    m = jnp.max(o, axis=1, keepdims=True)
    e = o - m
    lse = jnp.log(jnp.sum(jnp.exp(e), axis=1, keepdims=True))
    o_ref[...] = e - lse


_P_SPEC = pl.BlockSpec((2, BN, D), lambda i: (0, i, 0))
_DEG_SPEC = pl.BlockSpec((2, BN, DW), lambda i: (0, i, 0))
_H_SPEC = pl.BlockSpec((BN, D), lambda i: (i, 0))
_W_SPEC = pl.BlockSpec((D, D), lambda i: (0, 0))
_B_SPEC = pl.BlockSpec((1, D), lambda i: (0, 0))
_OUT_SHAPE = jax.ShapeDtypeStruct((N, D), jnp.float32)


_layer0 = pl.pallas_call(
    _layer0_body, grid=(N // BN,),
    in_specs=[_P_SPEC, _DEG_SPEC, _H_SPEC, _W_SPEC, _W_SPEC, _B_SPEC],
    out_specs=_H_SPEC, out_shape=_OUT_SHAPE)

_layer_res_relu = pl.pallas_call(
    functools.partial(_layer_res_body, relu=True), grid=(N // BN,),
    in_specs=[_P_SPEC, _DEG_SPEC, _H_SPEC, _W_SPEC, _W_SPEC, _B_SPEC,
              _W_SPEC, _B_SPEC],
    out_specs=_H_SPEC, out_shape=_OUT_SHAPE)

_layer_res = pl.pallas_call(
    functools.partial(_layer_res_body, relu=False), grid=(N // BN,),
    in_specs=[_P_SPEC, _DEG_SPEC, _H_SPEC, _W_SPEC, _W_SPEC, _B_SPEC,
              _W_SPEC, _B_SPEC],
    out_specs=_H_SPEC, out_shape=_OUT_SHAPE)

_mm2 = pl.pallas_call(
    _mm2_body, grid=(N // BN,),
    in_specs=[_H_SPEC, _H_SPEC, _W_SPEC, _W_SPEC],
    out_specs=_H_SPEC, out_shape=_OUT_SHAPE)

_final = pl.pallas_call(
    _final_body, grid=(N // BN,),
    in_specs=[_P_SPEC, _DEG_SPEC, _H_SPEC, _H_SPEC, _W_SPEC, _W_SPEC, _B_SPEC],
    out_specs=_H_SPEC, out_shape=_OUT_SHAPE)


def kernel(x, edge_index, params):
    src = edge_index[0]
    dst = edge_index[1]
    srcs = src.reshape(NW, NCHUNK, CH)
    dsts = dst.reshape(NW, NCHUNK, CH)
    zero = jnp.zeros((N, D), jnp.float32)
    ones_ch = jnp.ones((CH, D), jnp.float32)

    def agg(y):
        return _sc_agg(y, srcs, dsts, zero).reshape(2, N, D)

    degp = _sc_deg(dsts, ones_ch, zero).reshape(2, N, D)[:, :, :DW]
    p0 = agg(x)
    hs = {}
    for c in ("c1", "c2"):
        hs[c] = _layer0(p0, degp, x,
                        params[f"{c}_Wl0"], params[f"{c}_Wr0"],
                        params[f"{c}_bl0"].reshape(1, D))
    for i in range(1, 4):
        layer = _layer_res_relu if i < 3 else _layer_res
        for c in ("c1", "c2"):
            p = agg(hs[c])
            hs[c] = layer(p, degp, hs[c],
                          params[f"{c}_Wl{i}"], params[f"{c}_Wr{i}"],
                          params[f"{c}_bl{i}"].reshape(1, D),
                          params[f"{c}_Rw{i-1}"],
                          params[f"{c}_Rb{i-1}"].reshape(1, D))

    f_Wl = params["f_Wl"]
    f_Wr = params["f_Wr"]
    y = _mm2(hs["c1"], hs["c2"], f_Wl[:D], f_Wl[D:])
    pf = agg(y)
    return _final(pf, degp, hs["c1"], hs["c2"], f_Wr[:D], f_Wr[D:],
                  params["f_bl"].reshape(1, D))
